# baseline (speedup 1.0000x reference)
import functools

import jax
import jax.numpy as jnp
from jax import lax
from jax.experimental import pallas as pl
from jax.experimental.pallas import tpu as pltpu
from jax.experimental.pallas import tpu_sc as plsc

NC = 2
NS = 16
LANES = 16
NW = NC * NS
CHUNK = 128
NBUF = 6


def _mesh():
    return plsc.VectorSubcoreMesh(
        core_axis_name="c", subcore_axis_name="s",
        num_cores=NC, num_subcores=NS)


@functools.lru_cache(maxsize=None)
def _sc_scatter(npad, qc, rn, dh):
    rows_per = npad // NS

    @functools.partial(
        pl.kernel,
        out_type=jax.ShapeDtypeStruct((NC, npad, dh), jnp.float32),
        mesh=_mesh(),
        scratch_types=[
            pltpu.VMEM((qc, CHUNK), jnp.int32),
            pltpu.VMEM((qc, CHUNK), jnp.int32),
            *[pltpu.VMEM((CHUNK, dh), jnp.float32) for _ in range(NBUF)],
            pltpu.VMEM_SHARED((npad, dh), jnp.float32),
            *[pltpu.SemaphoreType.DMA for _ in range(NBUF + 1)],
        ],
        compiler_params=pltpu.CompilerParams(use_tc_tiling_on_sc=False),
    )
    def body(y_lo, y_hi, eidx_hbm, dst_hbm, out_hbm,
             eidx_v, dst_v, *rest):
        bufs = rest[:NBUF]
        agg_sh = rest[NBUF]
        sems = rest[NBUF + 1:NBUF + 1 + NBUF]
        ssem = rest[NBUF + 1 + NBUF]
        c = lax.axis_index("c")
        s = lax.axis_index("s")

        pltpu.sync_copy(eidx_hbm.at[s], eidx_v)
        pltpu.sync_copy(dst_hbm.at[s], dst_v)

        zbuf = bufs[0]

        def zrow(i, carry):
            for k in range(dh // LANES):
                zbuf[i, pl.ds(k * LANES, LANES)] = jnp.zeros(
                    (LANES,), jnp.float32)
            return carry
        lax.fori_loop(0, CHUNK, zrow, 0)
        for t in range(0, rows_per, CHUNK):
            w = min(CHUNK, rows_per - t)
            pltpu.sync_copy(
                zbuf.at[pl.ds(0, w)],
                agg_sh.at[pl.ds(s * rows_per + t, w)])
        plsc.subcore_barrier()

        def run(ytab):
            def group(j0, nb):
                gd = []
                for b in range(nb):
                    gd.append(pltpu.async_copy(
                        ytab.at[eidx_v.at[j0 + b]], bufs[b], sems[b]))
                for b in range(nb):
                    gd[b].wait()
                    pltpu.sync_copy(bufs[b], agg_sh.at[dst_v.at[j0 + b]],
                                    add=True)

            def outer(i, carry):
                group(i * NBUF, NBUF)
                return carry
            lax.fori_loop(0, qc // NBUF, outer, 0)
            if qc % NBUF:
                group((qc // NBUF) * NBUF, qc % NBUF)

        @pl.when(c == 0)
        def _():
            run(y_lo)

        @pl.when(c == 1)
        def _():
            run(y_hi)

        plsc.subcore_barrier()

        pltpu.sync_copy(agg_sh.at[pl.ds(s * rows_per, rows_per)],
                        out_hbm.at[c, pl.ds(s * rows_per, rows_per)])

    return body


def _rel_products(xb, w_lo_ref, w_hi_ref, y_ref, r):
    for i in range(r):
        y_ref[0, i] = jnp.dot(xb, w_lo_ref[i],
                              preferred_element_type=jnp.float32,
                              precision=lax.Precision.HIGHEST)
        y_ref[1, i] = jnp.dot(xb, w_hi_ref[i],
                              preferred_element_type=jnp.float32,
                              precision=lax.Precision.HIGHEST)


def _tc_rel_matmul(x, w_lo, w_hi):
    n, d = x.shape
    r = w_lo.shape[0]
    dh = d // 2
    bn = 1000

    def body(x_ref, wl_ref, wh_ref, y_ref):
        _rel_products(x_ref[...], wl_ref, wh_ref, y_ref, r)

    return pl.pallas_call(
        body,
        grid=(n // bn,),
        in_specs=[
            pl.BlockSpec((bn, d), lambda i: (i, 0)),
            pl.BlockSpec((r, d, dh), lambda i: (0, 0, 0)),
            pl.BlockSpec((r, d, dh), lambda i: (0, 0, 0)),
        ],
        out_specs=pl.BlockSpec((2, r, bn, dh), lambda i: (0, 0, i, 0)),
        out_shape=jax.ShapeDtypeStruct((2, r, n, dh), jnp.float32),
    )(x, w_lo, w_hi)


def _ln_block(a_ref, dg_ref, x_ref, w_ref, g_ref, b_ref):
    a = jnp.concatenate([a_ref[0], a_ref[1]], axis=1)
    dg = jnp.maximum(dg_ref[0, :, 0:1], 1.0)
    xb = x_ref[...]
    h = a / dg + jnp.dot(xb, w_ref[...],
                         preferred_element_type=jnp.float32,
                         precision=lax.Precision.HIGHEST)
    h = jnp.maximum(h, 0.0)
    t = h + xb
    mu = jnp.mean(t, axis=1, keepdims=True)
    var = jnp.mean((t - mu) ** 2, axis=1, keepdims=True)
    return ((t - mu) * lax.rsqrt(var + 1e-5) * g_ref[...] + b_ref[...])


def _tc_fused(agg2, deg2, x, w_self, gamma, beta, w_lo, w_hi):
    n, d = x.shape
    r = w_lo.shape[0]
    dh = d // 2
    bn = 1000

    def body(a_ref, dg_ref, x_ref, w_ref, g_ref, b_ref, wl_ref, wh_ref,
             o_ref, y_ref):
        xn = _ln_block(a_ref, dg_ref, x_ref, w_ref, g_ref, b_ref)
        o_ref[...] = xn
        _rel_products(xn, wl_ref, wh_ref, y_ref, r)

    return pl.pallas_call(
        body,
        grid=(n // bn,),
        in_specs=[
            pl.BlockSpec((NC, bn, dh), lambda i: (0, i, 0)),
            pl.BlockSpec((NC, bn, dh), lambda i: (0, i, 0)),
            pl.BlockSpec((bn, d), lambda i: (i, 0)),
            pl.BlockSpec((d, d), lambda i: (0, 0)),
            pl.BlockSpec((1, d), lambda i: (0, 0)),
            pl.BlockSpec((1, d), lambda i: (0, 0)),
            pl.BlockSpec((r, d, dh), lambda i: (0, 0, 0)),
            pl.BlockSpec((r, d, dh), lambda i: (0, 0, 0)),
        ],
        out_specs=[
            pl.BlockSpec((bn, d), lambda i: (i, 0)),
            pl.BlockSpec((2, r, bn, dh), lambda i: (0, 0, i, 0)),
        ],
        out_shape=[
            jax.ShapeDtypeStruct((n, d), jnp.float32),
            jax.ShapeDtypeStruct((2, r, n, dh), jnp.float32),
        ],
    )(agg2, deg2, x, w_self, gamma, beta, w_lo, w_hi)


def _tc_finalize(agg2, deg2, x, w_self, gamma, beta):
    n, d = x.shape
    dh = d // 2
    bn = 1000

    def body(a_ref, dg_ref, x_ref, w_ref, g_ref, b_ref, o_ref):
        o_ref[...] = _ln_block(a_ref, dg_ref, x_ref, w_ref, g_ref, b_ref)

    return pl.pallas_call(
        body,
        grid=(n // bn,),
        in_specs=[
            pl.BlockSpec((NC, bn, dh), lambda i: (0, i, 0)),
            pl.BlockSpec((NC, bn, dh), lambda i: (0, i, 0)),
            pl.BlockSpec((bn, d), lambda i: (i, 0)),
            pl.BlockSpec((d, d), lambda i: (0, 0)),
            pl.BlockSpec((1, d), lambda i: (0, 0)),
            pl.BlockSpec((1, d), lambda i: (0, 0)),
        ],
        out_specs=pl.BlockSpec((bn, d), lambda i: (i, 0)),
        out_shape=jax.ShapeDtypeStruct((n, d), jnp.float32),
    )(agg2, deg2, x, w_self, gamma, beta)


def kernel(annotation_ids, annotation_feature, annotation_edges,
           annotation_edges_type, W_rel, W_self, ln_gamma, ln_beta):
    x = annotation_feature
    n, d = x.shape
    num_layers, num_rel = W_rel.shape[0], W_rel.shape[1]
    e = annotation_edges.shape[1]
    src = annotation_edges[0]
    dst = annotation_edges[1]
    etype = annotation_edges_type
    dh = d // 2

    npad = ((n + 1 + NS - 1) // NS) * NS
    eidx = etype * n + src

    qc = -(-e // (NS * CHUNK))
    pad = NS * qc * CHUNK - e
    pad_idx = jnp.arange(pad, dtype=jnp.int32)
    eidx_r = jnp.concatenate(
        [eidx, pad_idx % (num_rel * n)]).reshape(NS, qc, CHUNK)
    dst_r = jnp.concatenate(
        [dst, n + pad_idx % (npad - n)]).reshape(NS, qc, CHUNK)

    scat = _sc_scatter(npad, qc, num_rel * n, dh)

    ones_tab = jnp.ones((num_rel * n, dh), jnp.float32)
    deg2 = scat(ones_tab, ones_tab, eidx_r, dst_r)

    w_lo = W_rel[:, :, :, :dh]
    w_hi = W_rel[:, :, :, dh:]
    y2 = _tc_rel_matmul(x, w_lo[0], w_hi[0]).reshape(2, num_rel * n, dh)
    for layer in range(num_layers):
        agg2 = scat(y2[0], y2[1], eidx_r, dst_r)
        gam = ln_gamma[layer].reshape(1, d)
        bet = ln_beta[layer].reshape(1, d)
        if layer + 1 < num_layers:
            x, y4 = _tc_fused(agg2, deg2, x, W_self[layer], gam, bet,
                              w_lo[layer + 1], w_hi[layer + 1])
            y2 = y4.reshape(2, num_rel * n, dh)
        else:
            x = _tc_finalize(agg2, deg2, x, W_self[layer], gam, bet)
    return x


# speedup vs baseline: 1.0258x; 1.0258x over previous
import functools

import jax
import jax.numpy as jnp
from jax import lax
from jax.experimental import pallas as pl
from jax.experimental.pallas import tpu as pltpu
from jax.experimental.pallas import tpu_sc as plsc

NC = 2
NS = 16
LANES = 16
NW = NC * NS
CHUNK = 128
NBUF = 6


def _mesh():
    return plsc.VectorSubcoreMesh(
        core_axis_name="c", subcore_axis_name="s",
        num_cores=NC, num_subcores=NS)


@functools.lru_cache(maxsize=None)
def _sc_scatter(npad, qc, rn, dh):
    rows_per = npad // NS

    @functools.partial(
        pl.kernel,
        out_type=jax.ShapeDtypeStruct((NC, npad, dh), jnp.float32),
        mesh=_mesh(),
        scratch_types=[
            pltpu.VMEM((qc, CHUNK), jnp.int32),
            pltpu.VMEM((qc, CHUNK), jnp.int32),
            *[pltpu.VMEM((CHUNK, dh), jnp.float32) for _ in range(NBUF)],
            pltpu.VMEM_SHARED((npad, dh), jnp.float32),
            *[pltpu.SemaphoreType.DMA for _ in range(NBUF + 1)],
        ],
        compiler_params=pltpu.CompilerParams(use_tc_tiling_on_sc=False),
    )
    def body(y_lo, y_hi, eidx_hbm, dst_hbm, out_hbm,
             eidx_v, dst_v, *rest):
        bufs = rest[:NBUF]
        agg_sh = rest[NBUF]
        sems = rest[NBUF + 1:NBUF + 1 + NBUF]
        ssem = rest[NBUF + 1 + NBUF]
        c = lax.axis_index("c")
        s = lax.axis_index("s")

        pltpu.sync_copy(eidx_hbm.at[s], eidx_v)
        pltpu.sync_copy(dst_hbm.at[s], dst_v)

        zbuf = bufs[0]

        def zrow(i, carry):
            for k in range(dh // LANES):
                zbuf[i, pl.ds(k * LANES, LANES)] = jnp.zeros(
                    (LANES,), jnp.float32)
            return carry
        lax.fori_loop(0, CHUNK, zrow, 0)
        for t in range(0, rows_per, CHUNK):
            w = min(CHUNK, rows_per - t)
            pltpu.sync_copy(
                zbuf.at[pl.ds(0, w)],
                agg_sh.at[pl.ds(s * rows_per + t, w)])
        plsc.subcore_barrier()

        def run(ytab):
            def group(j0, nb):
                gd = []
                for b in range(nb):
                    gd.append(pltpu.async_copy(
                        ytab.at[eidx_v.at[j0 + b]], bufs[b], sems[b]))
                for b in range(nb):
                    gd[b].wait()
                    pltpu.sync_copy(bufs[b], agg_sh.at[dst_v.at[j0 + b]],
                                    add=True)

            def outer(i, carry):
                group(i * NBUF, NBUF)
                return carry
            lax.fori_loop(0, qc // NBUF, outer, 0)
            if qc % NBUF:
                group((qc // NBUF) * NBUF, qc % NBUF)

        @pl.when(c == 0)
        def _():
            run(y_lo)

        @pl.when(c == 1)
        def _():
            run(y_hi)

        plsc.subcore_barrier()

        pltpu.sync_copy(agg_sh.at[pl.ds(s * rows_per, rows_per)],
                        out_hbm.at[c, pl.ds(s * rows_per, rows_per)])

    return body


def _rel_products(xb, w_lo_ref, w_hi_ref, y_ref, r):
    for i in range(r):
        y_ref[0, i] = jnp.dot(xb, w_lo_ref[i],
                              preferred_element_type=jnp.float32,
                              precision=lax.Precision.HIGHEST)
        y_ref[1, i] = jnp.dot(xb, w_hi_ref[i],
                              preferred_element_type=jnp.float32,
                              precision=lax.Precision.HIGHEST)


def _tc_rel_matmul(x, w_lo, w_hi):
    n, d = x.shape
    r = w_lo.shape[0]
    dh = d // 2
    bn = 1000

    def body(x_ref, wl_ref, wh_ref, y_ref):
        _rel_products(x_ref[...], wl_ref, wh_ref, y_ref, r)

    return pl.pallas_call(
        body,
        grid=(n // bn,),
        in_specs=[
            pl.BlockSpec((bn, d), lambda i: (i, 0)),
            pl.BlockSpec((r, d, dh), lambda i: (0, 0, 0)),
            pl.BlockSpec((r, d, dh), lambda i: (0, 0, 0)),
        ],
        out_specs=pl.BlockSpec((2, r, bn, dh), lambda i: (0, 0, i, 0)),
        out_shape=jax.ShapeDtypeStruct((2, r, n, dh), jnp.float32),
    )(x, w_lo, w_hi)


def _ln_block(a_ref, dg_ref, x_ref, w_ref, g_ref, b_ref):
    a = jnp.concatenate([a_ref[0], a_ref[1]], axis=1)
    dg = jnp.maximum(dg_ref[0, :, 0:1], 1.0)
    xb = x_ref[...]
    h = a / dg + jnp.dot(xb, w_ref[...],
                         preferred_element_type=jnp.float32,
                         precision=lax.Precision.HIGHEST)
    h = jnp.maximum(h, 0.0)
    t = h + xb
    mu = jnp.mean(t, axis=1, keepdims=True)
    var = jnp.mean((t - mu) ** 2, axis=1, keepdims=True)
    return ((t - mu) * lax.rsqrt(var + 1e-5) * g_ref[...] + b_ref[...])


def _tc_fused(agg2, deg2, x, w_self, gamma, beta, w_lo, w_hi):
    n, d = x.shape
    r = w_lo.shape[0]
    dh = d // 2
    bn = 1000

    def body(a_ref, dg_ref, x_ref, w_ref, g_ref, b_ref, wl_ref, wh_ref,
             o_ref, y_ref):
        xn = _ln_block(a_ref, dg_ref, x_ref, w_ref, g_ref, b_ref)
        o_ref[...] = xn
        _rel_products(xn, wl_ref, wh_ref, y_ref, r)

    return pl.pallas_call(
        body,
        grid=(n // bn,),
        in_specs=[
            pl.BlockSpec((NC, bn, dh), lambda i: (0, i, 0)),
            pl.BlockSpec((NC, bn, dh), lambda i: (0, i, 0)),
            pl.BlockSpec((bn, d), lambda i: (i, 0)),
            pl.BlockSpec((d, d), lambda i: (0, 0)),
            pl.BlockSpec((1, d), lambda i: (0, 0)),
            pl.BlockSpec((1, d), lambda i: (0, 0)),
            pl.BlockSpec((r, d, dh), lambda i: (0, 0, 0)),
            pl.BlockSpec((r, d, dh), lambda i: (0, 0, 0)),
        ],
        out_specs=[
            pl.BlockSpec((bn, d), lambda i: (i, 0)),
            pl.BlockSpec((2, r, bn, dh), lambda i: (0, 0, i, 0)),
        ],
        out_shape=[
            jax.ShapeDtypeStruct((n, d), jnp.float32),
            jax.ShapeDtypeStruct((2, r, n, dh), jnp.float32),
        ],
    )(agg2, deg2, x, w_self, gamma, beta, w_lo, w_hi)


def _tc_finalize(agg2, deg2, x, w_self, gamma, beta):
    n, d = x.shape
    dh = d // 2
    bn = 1000

    def body(a_ref, dg_ref, x_ref, w_ref, g_ref, b_ref, o_ref):
        o_ref[...] = _ln_block(a_ref, dg_ref, x_ref, w_ref, g_ref, b_ref)

    return pl.pallas_call(
        body,
        grid=(n // bn,),
        in_specs=[
            pl.BlockSpec((NC, bn, dh), lambda i: (0, i, 0)),
            pl.BlockSpec((NC, bn, dh), lambda i: (0, i, 0)),
            pl.BlockSpec((bn, d), lambda i: (i, 0)),
            pl.BlockSpec((d, d), lambda i: (0, 0)),
            pl.BlockSpec((1, d), lambda i: (0, 0)),
            pl.BlockSpec((1, d), lambda i: (0, 0)),
        ],
        out_specs=pl.BlockSpec((bn, d), lambda i: (i, 0)),
        out_shape=jax.ShapeDtypeStruct((n, d), jnp.float32),
    )(agg2, deg2, x, w_self, gamma, beta)


def kernel(annotation_ids, annotation_feature, annotation_edges,
           annotation_edges_type, W_rel, W_self, ln_gamma, ln_beta):
    x = annotation_feature
    n, d = x.shape
    num_layers, num_rel = W_rel.shape[0], W_rel.shape[1]
    e = annotation_edges.shape[1]
    src = annotation_edges[0]
    dst = annotation_edges[1]
    etype = annotation_edges_type
    dh = d // 2

    npad = ((n + 1 + NS - 1) // NS) * NS
    eidx = etype * n + src

    qc = -(-e // (NS * CHUNK))
    pad = NS * qc * CHUNK - e
    pad_idx = jnp.arange(pad, dtype=jnp.int32)
    eidx_r = jnp.concatenate(
        [eidx, pad_idx % (num_rel * n)]).reshape(NS, qc, CHUNK)
    dst_r = jnp.concatenate(
        [dst, n + pad_idx % (npad - n)]).reshape(NS, qc, CHUNK)

    scat = _sc_scatter(npad, qc, num_rel * n, dh)

    ones_tab = jnp.ones((num_rel * n, dh), jnp.float32)
    deg2 = scat(ones_tab, ones_tab, eidx_r, dst_r)

    w_lo = W_rel[:, :, :, :dh]
    w_hi = W_rel[:, :, :, dh:]
    for layer in range(num_layers):
        y2 = _tc_rel_matmul(x, w_lo[layer],
                            w_hi[layer]).reshape(2, num_rel * n, dh)
        agg2 = scat(y2[0], y2[1], eidx_r, dst_r)
        x = _tc_finalize(agg2, deg2, x, W_self[layer],
                         ln_gamma[layer].reshape(1, d),
                         ln_beta[layer].reshape(1, d))
    return x


# speedup vs baseline: 1.1985x; 1.1683x over previous
import functools

import jax
import jax.numpy as jnp
from jax import lax
from jax.experimental import pallas as pl
from jax.experimental.pallas import tpu as pltpu
from jax.experimental.pallas import tpu_sc as plsc

NC = 2
NS = 16
LANES = 16
NW = NC * NS
CHUNK = 128
NBUF = 6


def _mesh():
    return plsc.VectorSubcoreMesh(
        core_axis_name="c", subcore_axis_name="s",
        num_cores=NC, num_subcores=NS)


@functools.lru_cache(maxsize=None)
def _sc_scatter(npad, qc, rn, dh):
    rows_per = npad // NS

    @functools.partial(
        pl.kernel,
        out_type=jax.ShapeDtypeStruct((NC, npad, dh), jnp.float32),
        mesh=_mesh(),
        scratch_types=[
            pltpu.VMEM((qc, CHUNK), jnp.int32),
            pltpu.VMEM((qc, CHUNK), jnp.int32),
            *[pltpu.VMEM((CHUNK, dh), jnp.float32) for _ in range(NBUF)],
            pltpu.VMEM_SHARED((npad, dh), jnp.float32),
            *[pltpu.SemaphoreType.DMA for _ in range(NBUF + 1)],
        ],
        compiler_params=pltpu.CompilerParams(use_tc_tiling_on_sc=False),
    )
    def body(y_lo, y_hi, eidx_hbm, dst_hbm, out_hbm,
             eidx_v, dst_v, *rest):
        bufs = rest[:NBUF]
        agg_sh = rest[NBUF]
        sems = rest[NBUF + 1:NBUF + 1 + NBUF]
        ssem = rest[NBUF + 1 + NBUF]
        c = lax.axis_index("c")
        s = lax.axis_index("s")

        pltpu.sync_copy(eidx_hbm.at[s], eidx_v)
        pltpu.sync_copy(dst_hbm.at[s], dst_v)

        zbuf = bufs[0]

        def zrow(i, carry):
            for k in range(dh // LANES):
                zbuf[i, pl.ds(k * LANES, LANES)] = jnp.zeros(
                    (LANES,), jnp.float32)
            return carry
        lax.fori_loop(0, CHUNK, zrow, 0)
        for t in range(0, rows_per, CHUNK):
            w = min(CHUNK, rows_per - t)
            pltpu.sync_copy(
                zbuf.at[pl.ds(0, w)],
                agg_sh.at[pl.ds(s * rows_per + t, w)])
        plsc.subcore_barrier()

        def run(ytab):
            def group(j0, nb):
                gd = []
                for b in range(nb):
                    gd.append(pltpu.async_copy(
                        ytab.at[eidx_v.at[j0 + b]], bufs[b], sems[b]))
                for b in range(nb):
                    gd[b].wait()
                    pltpu.sync_copy(bufs[b], agg_sh.at[dst_v.at[j0 + b]],
                                    add=True)

            def outer(i, carry):
                group(i * NBUF, NBUF)
                return carry
            lax.fori_loop(0, qc // NBUF, outer, 0)
            if qc % NBUF:
                group((qc // NBUF) * NBUF, qc % NBUF)

        @pl.when(c == 0)
        def _():
            run(y_lo)

        @pl.when(c == 1)
        def _():
            run(y_hi)

        plsc.subcore_barrier()

        pltpu.sync_copy(agg_sh.at[pl.ds(s * rows_per, rows_per)],
                        out_hbm.at[c, pl.ds(s * rows_per, rows_per)])

    return body


def _tc_rel_matmul(x, w_rel):
    n, d = x.shape
    r = w_rel.shape[0]
    dh = d // 2
    bn = 1000

    def body(x_ref, w_ref, y_ref):
        xb = x_ref[...]
        for i in range(r):
            yi = jnp.dot(xb, w_ref[i],
                         preferred_element_type=jnp.float32,
                         precision=lax.Precision.HIGHEST)
            y_ref[0, i] = yi[:, :dh]
            y_ref[1, i] = yi[:, dh:]

    return pl.pallas_call(
        body,
        grid=(n // bn,),
        in_specs=[
            pl.BlockSpec((bn, d), lambda i: (i, 0)),
            pl.BlockSpec((r, d, d), lambda i: (0, 0, 0)),
        ],
        out_specs=pl.BlockSpec((2, r, bn, dh), lambda i: (0, 0, i, 0)),
        out_shape=jax.ShapeDtypeStruct((2, r, n, dh), jnp.float32),
    )(x, w_rel)


def _ln_block(a_ref, dg_ref, x_ref, w_ref, g_ref, b_ref):
    a = jnp.concatenate([a_ref[0], a_ref[1]], axis=1)
    dg = jnp.maximum(dg_ref[0, :, 0:1], 1.0)
    xb = x_ref[...]
    h = a / dg + jnp.dot(xb, w_ref[...],
                         preferred_element_type=jnp.float32,
                         precision=lax.Precision.HIGHEST)
    h = jnp.maximum(h, 0.0)
    t = h + xb
    mu = jnp.mean(t, axis=1, keepdims=True)
    var = jnp.mean((t - mu) ** 2, axis=1, keepdims=True)
    return ((t - mu) * lax.rsqrt(var + 1e-5) * g_ref[...] + b_ref[...])


def _tc_finalize(agg2, deg2, x, w_self, gamma, beta):
    n, d = x.shape
    dh = d // 2
    bn = 1000

    def body(a_ref, dg_ref, x_ref, w_ref, g_ref, b_ref, o_ref):
        o_ref[...] = _ln_block(a_ref, dg_ref, x_ref, w_ref, g_ref, b_ref)

    return pl.pallas_call(
        body,
        grid=(n // bn,),
        in_specs=[
            pl.BlockSpec((NC, bn, dh), lambda i: (0, i, 0)),
            pl.BlockSpec((NC, bn, dh), lambda i: (0, i, 0)),
            pl.BlockSpec((bn, d), lambda i: (i, 0)),
            pl.BlockSpec((d, d), lambda i: (0, 0)),
            pl.BlockSpec((1, d), lambda i: (0, 0)),
            pl.BlockSpec((1, d), lambda i: (0, 0)),
        ],
        out_specs=pl.BlockSpec((bn, d), lambda i: (i, 0)),
        out_shape=jax.ShapeDtypeStruct((n, d), jnp.float32),
    )(agg2, deg2, x, w_self, gamma, beta)


def kernel(annotation_ids, annotation_feature, annotation_edges,
           annotation_edges_type, W_rel, W_self, ln_gamma, ln_beta):
    x = annotation_feature
    n, d = x.shape
    num_layers, num_rel = W_rel.shape[0], W_rel.shape[1]
    e = annotation_edges.shape[1]
    src = annotation_edges[0]
    dst = annotation_edges[1]
    etype = annotation_edges_type
    dh = d // 2

    npad = ((n + 1 + NS - 1) // NS) * NS
    eidx = etype * n + src

    qc = -(-e // (NS * CHUNK))
    pad = NS * qc * CHUNK - e
    pad_idx = jnp.arange(pad, dtype=jnp.int32)
    eidx_r = jnp.concatenate(
        [eidx, pad_idx % (num_rel * n)]).reshape(NS, qc, CHUNK)
    dst_r = jnp.concatenate(
        [dst, n + pad_idx % (npad - n)]).reshape(NS, qc, CHUNK)

    scat = _sc_scatter(npad, qc, num_rel * n, dh)

    ones_tab = jnp.ones((num_rel * n, dh), jnp.float32)
    deg2 = scat(ones_tab, ones_tab, eidx_r, dst_r)

    for layer in range(num_layers):
        y2 = _tc_rel_matmul(x, W_rel[layer]).reshape(2, num_rel * n, dh)
        agg2 = scat(y2[0], y2[1], eidx_r, dst_r)
        x = _tc_finalize(agg2, deg2, x, W_self[layer],
                         ln_gamma[layer].reshape(1, d),
                         ln_beta[layer].reshape(1, d))
    return x


# speedup vs baseline: 1.3087x; 1.0920x over previous
import functools

import jax
import jax.numpy as jnp
from jax import lax
from jax.experimental import pallas as pl
from jax.experimental.pallas import tpu as pltpu
from jax.experimental.pallas import tpu_sc as plsc

NC = 2
NS = 16
LANES = 16
NW = NC * NS
CHUNK = 128
NBUF = 6


def _mesh():
    return plsc.VectorSubcoreMesh(
        core_axis_name="c", subcore_axis_name="s",
        num_cores=NC, num_subcores=NS)


@functools.lru_cache(maxsize=None)
def _sc_scatter(npad, qc, rn, dh):
    rows_per = npad // NS

    @functools.partial(
        pl.kernel,
        out_type=jax.ShapeDtypeStruct((NC, npad, dh), jnp.float32),
        mesh=_mesh(),
        scratch_types=[
            pltpu.VMEM((qc, CHUNK), jnp.int32),
            pltpu.VMEM((qc, CHUNK), jnp.int32),
            *[pltpu.VMEM((CHUNK, dh), jnp.float32) for _ in range(NBUF)],
            pltpu.VMEM_SHARED((npad, dh), jnp.float32),
            *[pltpu.SemaphoreType.DMA for _ in range(NBUF + 1)],
        ],
        compiler_params=pltpu.CompilerParams(use_tc_tiling_on_sc=False),
    )
    def body(y_lo, y_hi, eidx_hbm, dst_hbm, out_hbm,
             eidx_v, dst_v, *rest):
        bufs = rest[:NBUF]
        agg_sh = rest[NBUF]
        sems = rest[NBUF + 1:NBUF + 1 + NBUF]
        ssem = rest[NBUF + 1 + NBUF]
        c = lax.axis_index("c")
        s = lax.axis_index("s")

        pltpu.sync_copy(eidx_hbm.at[s], eidx_v)
        pltpu.sync_copy(dst_hbm.at[s], dst_v)

        zbuf = bufs[0]

        def zrow(i, carry):
            for k in range(dh // LANES):
                zbuf[i, pl.ds(k * LANES, LANES)] = jnp.zeros(
                    (LANES,), jnp.float32)
            return carry
        lax.fori_loop(0, CHUNK, zrow, 0)
        for t in range(0, rows_per, CHUNK):
            w = min(CHUNK, rows_per - t)
            pltpu.sync_copy(
                zbuf.at[pl.ds(0, w)],
                agg_sh.at[pl.ds(s * rows_per + t, w)])
        plsc.subcore_barrier()

        def run(ytab):
            def group(j0, nb):
                gd = []
                for b in range(nb):
                    gd.append(pltpu.async_copy(
                        ytab.at[eidx_v.at[j0 + b]], bufs[b], sems[b]))
                for b in range(nb):
                    gd[b].wait()
                    pltpu.sync_copy(bufs[b], agg_sh.at[dst_v.at[j0 + b]],
                                    add=True)

            def outer(i, carry):
                group(i * NBUF, NBUF)
                return carry
            lax.fori_loop(0, qc // NBUF, outer, 0)
            if qc % NBUF:
                group((qc // NBUF) * NBUF, qc % NBUF)

        @pl.when(c == 0)
        def _():
            run(y_lo)

        @pl.when(c == 1)
        def _():
            run(y_hi)

        plsc.subcore_barrier()

        pltpu.sync_copy(agg_sh.at[pl.ds(s * rows_per, rows_per)],
                        out_hbm.at[c, pl.ds(s * rows_per, rows_per)])

    return body


def _tc_rel_matmul(x, w_rel):
    n, d = x.shape
    r = w_rel.shape[0]
    dh = d // 2
    bn = 1000

    def body(x_ref, w_ref, y_ref):
        xb = x_ref[...]
        for i in range(r):
            yi = jnp.dot(xb, w_ref[i],
                         preferred_element_type=jnp.float32)
            y_ref[0, i] = yi[:, :dh]
            y_ref[1, i] = yi[:, dh:]

    return pl.pallas_call(
        body,
        grid=(n // bn,),
        in_specs=[
            pl.BlockSpec((bn, d), lambda i: (i, 0)),
            pl.BlockSpec((r, d, d), lambda i: (0, 0, 0)),
        ],
        out_specs=pl.BlockSpec((2, r, bn, dh), lambda i: (0, 0, i, 0)),
        out_shape=jax.ShapeDtypeStruct((2, r, n, dh), jnp.float32),
    )(x, w_rel)


def _ln_block(a_ref, dg_ref, x_ref, w_ref, g_ref, b_ref):
    a = jnp.concatenate([a_ref[0], a_ref[1]], axis=1)
    dg = jnp.maximum(dg_ref[0, :, 0:1], 1.0)
    xb = x_ref[...]
    h = a / dg + jnp.dot(xb, w_ref[...],
                         preferred_element_type=jnp.float32)
    h = jnp.maximum(h, 0.0)
    t = h + xb
    mu = jnp.mean(t, axis=1, keepdims=True)
    var = jnp.mean((t - mu) ** 2, axis=1, keepdims=True)
    return ((t - mu) * lax.rsqrt(var + 1e-5) * g_ref[...] + b_ref[...])


def _tc_finalize(agg2, deg2, x, w_self, gamma, beta):
    n, d = x.shape
    dh = d // 2
    bn = 1000

    def body(a_ref, dg_ref, x_ref, w_ref, g_ref, b_ref, o_ref):
        o_ref[...] = _ln_block(a_ref, dg_ref, x_ref, w_ref, g_ref, b_ref)

    return pl.pallas_call(
        body,
        grid=(n // bn,),
        in_specs=[
            pl.BlockSpec((NC, bn, dh), lambda i: (0, i, 0)),
            pl.BlockSpec((NC, bn, dh), lambda i: (0, i, 0)),
            pl.BlockSpec((bn, d), lambda i: (i, 0)),
            pl.BlockSpec((d, d), lambda i: (0, 0)),
            pl.BlockSpec((1, d), lambda i: (0, 0)),
            pl.BlockSpec((1, d), lambda i: (0, 0)),
        ],
        out_specs=pl.BlockSpec((bn, d), lambda i: (i, 0)),
        out_shape=jax.ShapeDtypeStruct((n, d), jnp.float32),
    )(agg2, deg2, x, w_self, gamma, beta)


def kernel(annotation_ids, annotation_feature, annotation_edges,
           annotation_edges_type, W_rel, W_self, ln_gamma, ln_beta):
    x = annotation_feature
    n, d = x.shape
    num_layers, num_rel = W_rel.shape[0], W_rel.shape[1]
    e = annotation_edges.shape[1]
    src = annotation_edges[0]
    dst = annotation_edges[1]
    etype = annotation_edges_type
    dh = d // 2

    npad = ((n + 1 + NS - 1) // NS) * NS
    eidx = etype * n + src

    qc = -(-e // (NS * CHUNK))
    pad = NS * qc * CHUNK - e
    pad_idx = jnp.arange(pad, dtype=jnp.int32)
    eidx_r = jnp.concatenate(
        [eidx, pad_idx % (num_rel * n)]).reshape(NS, qc, CHUNK)
    dst_r = jnp.concatenate(
        [dst, n + pad_idx % (npad - n)]).reshape(NS, qc, CHUNK)

    scat = _sc_scatter(npad, qc, num_rel * n, dh)

    ones_tab = jnp.ones((num_rel * n, dh), jnp.float32)
    deg2 = scat(ones_tab, ones_tab, eidx_r, dst_r)

    for layer in range(num_layers):
        y2 = _tc_rel_matmul(x, W_rel[layer]).reshape(2, num_rel * n, dh)
        agg2 = scat(y2[0], y2[1], eidx_r, dst_r)
        x = _tc_finalize(agg2, deg2, x, W_self[layer],
                         ln_gamma[layer].reshape(1, d),
                         ln_beta[layer].reshape(1, d))
    return x


# speedup vs baseline: 1.3222x; 1.0103x over previous
import functools

import jax
import jax.numpy as jnp
from jax import lax
from jax.experimental import pallas as pl
from jax.experimental.pallas import tpu as pltpu
from jax.experimental.pallas import tpu_sc as plsc

NC = 2
NS = 16
LANES = 16
NW = NC * NS
CHUNK = 128
NBUF = 6


def _mesh():
    return plsc.VectorSubcoreMesh(
        core_axis_name="c", subcore_axis_name="s",
        num_cores=NC, num_subcores=NS)


@functools.lru_cache(maxsize=None)
def _sc_scatter(npad, qc, rn, dh):
    rows_per = npad // NS

    @functools.partial(
        pl.kernel,
        out_type=jax.ShapeDtypeStruct((NC, npad, dh), jnp.float32),
        mesh=_mesh(),
        scratch_types=[
            pltpu.VMEM((qc, CHUNK), jnp.int32),
            pltpu.VMEM((qc, CHUNK), jnp.int32),
            *[pltpu.VMEM((CHUNK, dh), jnp.float32) for _ in range(NBUF)],
            pltpu.VMEM_SHARED((npad, dh), jnp.float32),
            *[pltpu.SemaphoreType.DMA for _ in range(NBUF + 1)],
        ],
        compiler_params=pltpu.CompilerParams(use_tc_tiling_on_sc=False),
    )
    def body(y_lo, y_hi, eidx_hbm, dst_hbm, out_hbm,
             eidx_v, dst_v, *rest):
        bufs = rest[:NBUF]
        agg_sh = rest[NBUF]
        sems = rest[NBUF + 1:NBUF + 1 + NBUF]
        ssem = rest[NBUF + 1 + NBUF]
        c = lax.axis_index("c")
        s = lax.axis_index("s")

        pltpu.sync_copy(eidx_hbm.at[s], eidx_v)
        pltpu.sync_copy(dst_hbm.at[s], dst_v)

        zbuf = bufs[0]

        def zrow(i, carry):
            for k in range(dh // LANES):
                zbuf[i, pl.ds(k * LANES, LANES)] = jnp.zeros(
                    (LANES,), jnp.float32)
            return carry
        lax.fori_loop(0, CHUNK, zrow, 0)
        for t in range(0, rows_per, CHUNK):
            w = min(CHUNK, rows_per - t)
            pltpu.sync_copy(
                zbuf.at[pl.ds(0, w)],
                agg_sh.at[pl.ds(s * rows_per + t, w)])
        plsc.subcore_barrier()

        def run(ytab):
            def group(j0, nb):
                gd = []
                for b in range(nb):
                    gd.append(pltpu.async_copy(
                        ytab.at[eidx_v.at[j0 + b]], bufs[b], sems[b]))
                for b in range(nb):
                    gd[b].wait()
                    pltpu.sync_copy(bufs[b], agg_sh.at[dst_v.at[j0 + b]],
                                    add=True)

            def outer(i, carry):
                group(i * NBUF, NBUF)
                return carry
            lax.fori_loop(0, qc // NBUF, outer, 0)
            if qc % NBUF:
                group((qc // NBUF) * NBUF, qc % NBUF)

        @pl.when(c == 0)
        def _():
            run(y_lo)

        @pl.when(c == 1)
        def _():
            run(y_hi)

        plsc.subcore_barrier()

        pltpu.sync_copy(agg_sh.at[pl.ds(s * rows_per, rows_per)],
                        out_hbm.at[c, pl.ds(s * rows_per, rows_per)])

    return body


def _tc_rel_matmul(x, w_rel):
    n, d = x.shape
    r = w_rel.shape[0]
    dh = d // 2
    bn = 2000

    def body(x_ref, w_ref, y_ref):
        xb = x_ref[...]
        for i in range(r):
            yi = jnp.dot(xb, w_ref[i],
                         preferred_element_type=jnp.float32)
            y_ref[0, i] = yi[:, :dh]
            y_ref[1, i] = yi[:, dh:]

    return pl.pallas_call(
        body,
        grid=(n // bn,),
        in_specs=[
            pl.BlockSpec((bn, d), lambda i: (i, 0)),
            pl.BlockSpec((r, d, d), lambda i: (0, 0, 0)),
        ],
        out_specs=pl.BlockSpec((2, r, bn, dh), lambda i: (0, 0, i, 0)),
        out_shape=jax.ShapeDtypeStruct((2, r, n, dh), jnp.float32),
    )(x, w_rel)


def _ln_block(a_ref, dg_ref, x_ref, w_ref, g_ref, b_ref):
    a = jnp.concatenate([a_ref[0], a_ref[1]], axis=1)
    dg = jnp.maximum(dg_ref[0, :, 0:1], 1.0)
    xb = x_ref[...]
    h = a / dg + jnp.dot(xb, w_ref[...],
                         preferred_element_type=jnp.float32)
    h = jnp.maximum(h, 0.0)
    t = h + xb
    mu = jnp.mean(t, axis=1, keepdims=True)
    var = jnp.mean((t - mu) ** 2, axis=1, keepdims=True)
    return ((t - mu) * lax.rsqrt(var + 1e-5) * g_ref[...] + b_ref[...])


def _tc_finalize(agg2, deg2, x, w_self, gamma, beta):
    n, d = x.shape
    dh = d // 2
    bn = 2000

    def body(a_ref, dg_ref, x_ref, w_ref, g_ref, b_ref, o_ref):
        o_ref[...] = _ln_block(a_ref, dg_ref, x_ref, w_ref, g_ref, b_ref)

    return pl.pallas_call(
        body,
        grid=(n // bn,),
        in_specs=[
            pl.BlockSpec((NC, bn, dh), lambda i: (0, i, 0)),
            pl.BlockSpec((NC, bn, dh), lambda i: (0, i, 0)),
            pl.BlockSpec((bn, d), lambda i: (i, 0)),
            pl.BlockSpec((d, d), lambda i: (0, 0)),
            pl.BlockSpec((1, d), lambda i: (0, 0)),
            pl.BlockSpec((1, d), lambda i: (0, 0)),
        ],
        out_specs=pl.BlockSpec((bn, d), lambda i: (i, 0)),
        out_shape=jax.ShapeDtypeStruct((n, d), jnp.float32),
    )(agg2, deg2, x, w_self, gamma, beta)


def kernel(annotation_ids, annotation_feature, annotation_edges,
           annotation_edges_type, W_rel, W_self, ln_gamma, ln_beta):
    x = annotation_feature
    n, d = x.shape
    num_layers, num_rel = W_rel.shape[0], W_rel.shape[1]
    e = annotation_edges.shape[1]
    src = annotation_edges[0]
    dst = annotation_edges[1]
    etype = annotation_edges_type
    dh = d // 2

    npad = ((n + 1 + NS - 1) // NS) * NS
    eidx = etype * n + src

    qc = -(-e // (NS * CHUNK))
    pad = NS * qc * CHUNK - e
    pad_idx = jnp.arange(pad, dtype=jnp.int32)
    eidx_r = jnp.concatenate(
        [eidx, pad_idx % (num_rel * n)]).reshape(NS, qc, CHUNK)
    dst_r = jnp.concatenate(
        [dst, n + pad_idx % (npad - n)]).reshape(NS, qc, CHUNK)

    scat = _sc_scatter(npad, qc, num_rel * n, dh)

    ones_tab = jnp.ones((num_rel * n, dh), jnp.float32)
    deg2 = scat(ones_tab, ones_tab, eidx_r, dst_r)

    for layer in range(num_layers):
        y2 = _tc_rel_matmul(x, W_rel[layer]).reshape(2, num_rel * n, dh)
        agg2 = scat(y2[0], y2[1], eidx_r, dst_r)
        x = _tc_finalize(agg2, deg2, x, W_self[layer],
                         ln_gamma[layer].reshape(1, d),
                         ln_beta[layer].reshape(1, d))
    return x
